# Initial kernel scaffold; baseline (speedup 1.0000x reference)
import jax
import jax.numpy as jnp
from jax.experimental import pallas as pl


def kernel(x, edge_index, input_W, input_b, msg_W, msg_b, upd_W, upd_b, gru_Wih, gru_Whh, gru_bih, gru_bhh, mu_W, mu_b, lv_W, lv_b, gmu_W1, gmu_b1, gmu_W2, gmu_b2, glv_W1, glv_b1, glv_W2, glv_b2):
    raise NotImplementedError("write your pallas kernel here")


# speedup vs baseline: 3.0188x; 3.0188x over previous
import functools

import jax
import jax.numpy as jnp
from jax import lax
from jax.experimental import pallas as pl
from jax.experimental.pallas import tpu as pltpu
from jax.experimental.pallas import tpu_sc as plsc

N = 50000
E = 800000
IN = 128
S = 64
Z = 32
G = 16
R = 6

NC = 2
NS = 16
HALF = 25000
HALF_PAD = 25600
TROWS = HALF_PAD // NS
TRASH = HALF_PAD - 1
CH = 128
EPT = E // NS
NFULL = EPT // CH
TAIL = EPT - NFULL * CH
ZROWS = 160

_sc_mesh = plsc.VectorSubcoreMesh(core_axis_name="c", subcore_axis_name="s")


@functools.partial(
    pl.kernel,
    out_type=jax.ShapeDtypeStruct((NC * HALF_PAD, S), jnp.float32),
    mesh=_sc_mesh,
    scratch_types=[
        pltpu.VMEM((CH,), jnp.int32),
        pltpu.VMEM((CH,), jnp.int32),
        pltpu.VMEM((CH, S), jnp.float32),
        pltpu.VMEM((ZROWS, S), jnp.float32),
        pltpu.VMEM_SHARED((HALF_PAD, S), jnp.float32),
        pltpu.SemaphoreType.DMA,
    ],
    compiler_params=pltpu.CompilerParams(use_tc_tiling_on_sc=False),
)
def _sc_agg(m_hbm, src_hbm, dst_hbm, out_hbm, srcv, dstv, rows, zbuf, aggsh,
            sem):
    c = lax.axis_index("c")
    s = lax.axis_index("s")
    base_node = c * HALF

    zero16 = jnp.zeros((16,), jnp.float32)

    def _zero_row(i, carry):
        for j in range(S // 16):
            zbuf[i, pl.ds(j * 16, 16)] = zero16
        return carry

    lax.fori_loop(0, ZROWS, _zero_row, 0)

    def _zero_copy(k, carry):
        pltpu.sync_copy(zbuf, aggsh.at[pl.ds(s * TROWS + k * ZROWS, ZROWS)])
        return carry

    lax.fori_loop(0, TROWS // ZROWS, _zero_copy, 0)
    plsc.subcore_barrier()

    def _chunk(eb, ch):
        pltpu.sync_copy(src_hbm.at[pl.ds(eb, ch)], srcv.at[pl.ds(0, ch)])
        pltpu.sync_copy(dst_hbm.at[pl.ds(eb, ch)], dstv.at[pl.ds(0, ch)])

        def _remap(j, carry):
            d = dstv[pl.ds(j * 16, 16)]
            loc = d - base_node
            keep = (loc >= 0) & (loc < HALF)
            dstv[pl.ds(j * 16, 16)] = jnp.where(keep, loc, TRASH)
            return carry

        lax.fori_loop(0, ch // 16, _remap, 0)
        if ch == CH:
            pltpu.async_copy(m_hbm.at[srcv], rows, sem).wait()
            pltpu.sync_copy(rows, aggsh.at[dstv], add=True)
        else:
            pltpu.async_copy(
                m_hbm.at[srcv.at[pl.ds(0, ch)]], rows.at[pl.ds(0, ch)], sem
            ).wait()
            pltpu.sync_copy(
                rows.at[pl.ds(0, ch)], aggsh.at[dstv.at[pl.ds(0, ch)]],
                add=True)

    def _edge_body(i, carry):
        _chunk(s * EPT + i * CH, CH)
        return carry

    lax.fori_loop(0, NFULL, _edge_body, 0)
    if TAIL:
        _chunk(s * EPT + NFULL * CH, TAIL)

    plsc.subcore_barrier()
    pltpu.sync_copy(
        aggsh.at[pl.ds(s * TROWS, TROWS)],
        out_hbm.at[pl.ds(c * HALF_PAD + s * TROWS, TROWS)],
    )


BN = 2000
NBLK = N // BN


def _tc_init_body(x_ref, wi_ref, bi_ref, wm_ref, bm_ref, h_ref, m_ref):
    h = jnp.maximum(
        jnp.dot(x_ref[...], wi_ref[...],
                preferred_element_type=jnp.float32) + bi_ref[...], 0.0)
    h_ref[...] = h
    m_ref[...] = jnp.maximum(
        jnp.dot(h, wm_ref[...], preferred_element_type=jnp.float32)
        + bm_ref[...], 0.0)


def _gru_from_agg(agg, h, wu, bu, wr, br, wz, bz, wn, bn, hr, hz, hn_w,
                  bhr, bhz, bhn):
    msg = jnp.maximum(
        jnp.dot(agg, wu, preferred_element_type=jnp.float32) + bu, 0.0)
    g_r = jax.nn.sigmoid(
        jnp.dot(msg, wr, preferred_element_type=jnp.float32) + br
        + jnp.dot(h, hr, preferred_element_type=jnp.float32) + bhr)
    g_z = jax.nn.sigmoid(
        jnp.dot(msg, wz, preferred_element_type=jnp.float32) + bz
        + jnp.dot(h, hz, preferred_element_type=jnp.float32) + bhz)
    g_n = jnp.tanh(
        jnp.dot(msg, wn, preferred_element_type=jnp.float32) + bn
        + g_r * (jnp.dot(h, hn_w, preferred_element_type=jnp.float32) + bhn))
    return (1.0 - g_z) * g_n + g_z * h


def _tc_round_body(agg_ref, h_ref, wu_ref, bu_ref, wr_ref, br_ref, wz_ref,
                   bz_ref, wn_ref, bn_ref, hr_ref, hz_ref, hn_ref, bhr_ref,
                   bhz_ref, bhn_ref, wm_ref, bm_ref, hout_ref, mout_ref):
    hn = _gru_from_agg(agg_ref[...], h_ref[...], wu_ref[...], bu_ref[...],
                       wr_ref[...], br_ref[...], wz_ref[...], bz_ref[...],
                       wn_ref[...], bn_ref[...], hr_ref[...], hz_ref[...],
                       hn_ref[...], bhr_ref[...], bhz_ref[...], bhn_ref[...])
    hout_ref[...] = hn
    mout_ref[...] = jnp.maximum(
        jnp.dot(hn, wm_ref[...], preferred_element_type=jnp.float32)
        + bm_ref[...], 0.0)


def _tc_last_body(agg_ref, h_ref, wu_ref, bu_ref, wr_ref, br_ref, wz_ref,
                  bz_ref, wn_ref, bn_ref, hr_ref, hz_ref, hn_ref, bhr_ref,
                  bhz_ref, bhn_ref, muw_ref, mub_ref, lvw_ref, lvb_ref,
                  gm1_ref, gb1_ref, gm2_ref, gb2_ref, gl1_ref, gc1_ref,
                  gl2_ref, gc2_ref, mu_ref, lv_ref, mug_ref, lvg_ref,
                  sum_ref):
    i = pl.program_id(0)
    hn = _gru_from_agg(agg_ref[...], h_ref[...], wu_ref[...], bu_ref[...],
                       wr_ref[...], br_ref[...], wz_ref[...], bz_ref[...],
                       wn_ref[...], bn_ref[...], hr_ref[...], hz_ref[...],
                       hn_ref[...], bhr_ref[...], bhz_ref[...], bhn_ref[...])
    mu_ref[...] = jnp.dot(
        hn, muw_ref[...], preferred_element_type=jnp.float32) + mub_ref[...]
    lv_ref[...] = jnp.dot(
        hn, lvw_ref[...], preferred_element_type=jnp.float32) + lvb_ref[...]

    bsum = jnp.sum(hn, axis=0, keepdims=True)

    @pl.when(i == 0)
    def _():
        sum_ref[...] = jnp.zeros_like(sum_ref)

    sum_ref[...] += jnp.broadcast_to(bsum, sum_ref.shape)

    @pl.when(i == NBLK - 1)
    def _():
        g = sum_ref[0:1, :] * (1.0 / N)
        gmu = jnp.dot(
            jnp.maximum(
                jnp.dot(g, gm1_ref[...],
                        preferred_element_type=jnp.float32) + gb1_ref[...],
                0.0),
            gm2_ref[...], preferred_element_type=jnp.float32) + gb2_ref[...]
        glv = jnp.dot(
            jnp.maximum(
                jnp.dot(g, gl1_ref[...],
                        preferred_element_type=jnp.float32) + gc1_ref[...],
                0.0),
            gl2_ref[...], preferred_element_type=jnp.float32) + gc2_ref[...]
        mug_ref[...] = jnp.broadcast_to(gmu, mug_ref.shape)
        lvg_ref[...] = jnp.broadcast_to(glv, lvg_ref.shape)


def _row_spec(cols):
    return pl.BlockSpec((BN, cols), lambda i: (i, 0))


def _w_spec(shape):
    nd = len(shape)
    return pl.BlockSpec(shape, lambda i, _nd=nd: (0,) * _nd)


def kernel(x, edge_index, input_W, input_b, msg_W, msg_b, upd_W, upd_b,
           gru_Wih, gru_Whh, gru_bih, gru_bhh, mu_W, mu_b, lv_W, lv_b,
           gmu_W1, gmu_b1, gmu_W2, gmu_b2, glv_W1, glv_b1, glv_W2, glv_b2):
    f32 = jnp.float32
    src = edge_index[0]
    dst = edge_index[1]

    wih_t = gru_Wih.T
    whh_t = gru_Whh.T
    wr, wz, wn = wih_t[:, :S], wih_t[:, S:2 * S], wih_t[:, 2 * S:]
    hr, hz, hn = whh_t[:, :S], whh_t[:, S:2 * S], whh_t[:, 2 * S:]
    br = gru_bih[:S].reshape(1, S)
    bz = gru_bih[S:2 * S].reshape(1, S)
    bn = gru_bih[2 * S:].reshape(1, S)
    bhr = gru_bhh[:S].reshape(1, S)
    bhz = gru_bhh[S:2 * S].reshape(1, S)
    bhn = gru_bhh[2 * S:].reshape(1, S)

    h, m = pl.pallas_call(
        _tc_init_body,
        grid=(NBLK,),
        in_specs=[
            _row_spec(IN),
            _w_spec((IN, S)), _w_spec((1, S)),
            _w_spec((S, S)), _w_spec((1, S)),
        ],
        out_specs=[_row_spec(S), _row_spec(S)],
        out_shape=[
            jax.ShapeDtypeStruct((N, S), f32),
            jax.ShapeDtypeStruct((N, S), f32),
        ],
    )(x, input_W, input_b.reshape(1, S), msg_W[0], msg_b[0].reshape(1, S))

    round_specs = (
        [_row_spec(S), _row_spec(S)]
        + [_w_spec((S, S)), _w_spec((1, S))] * 4
        + [_w_spec((S, S))] * 3
        + [_w_spec((1, S))] * 3
    )

    for r in range(R):
        agg_pad = _sc_agg(m, src, dst)
        agg = jnp.concatenate(
            [agg_pad[:HALF], agg_pad[HALF_PAD:HALF_PAD + HALF]], axis=0)
        round_args = (
            agg, h,
            upd_W[r], upd_b[r].reshape(1, S),
            wr, br, wz, bz, wn, bn,
            hr, hz, hn, bhr, bhz, bhn,
        )
        if r < R - 1:
            h, m = pl.pallas_call(
                _tc_round_body,
                grid=(NBLK,),
                in_specs=round_specs + [_w_spec((S, S)), _w_spec((1, S))],
                out_specs=[_row_spec(S), _row_spec(S)],
                out_shape=[
                    jax.ShapeDtypeStruct((N, S), f32),
                    jax.ShapeDtypeStruct((N, S), f32),
                ],
            )(*round_args, msg_W[r + 1], msg_b[r + 1].reshape(1, S))
        else:
            mu_node, lv_node, mu_g8, lv_g8 = pl.pallas_call(
                _tc_last_body,
                grid=(NBLK,),
                in_specs=round_specs + [
                    _w_spec((S, Z)), _w_spec((1, Z)),
                    _w_spec((S, Z)), _w_spec((1, Z)),
                    _w_spec((S, S)), _w_spec((1, S)),
                    _w_spec((S, G)), _w_spec((1, G)),
                    _w_spec((S, S)), _w_spec((1, S)),
                    _w_spec((S, G)), _w_spec((1, G)),
                ],
                out_specs=[
                    _row_spec(Z), _row_spec(Z),
                    pl.BlockSpec((8, G), lambda i: (0, 0)),
                    pl.BlockSpec((8, G), lambda i: (0, 0)),
                ],
                out_shape=[
                    jax.ShapeDtypeStruct((N, Z), f32),
                    jax.ShapeDtypeStruct((N, Z), f32),
                    jax.ShapeDtypeStruct((8, G), f32),
                    jax.ShapeDtypeStruct((8, G), f32),
                ],
                scratch_shapes=[pltpu.VMEM((8, S), f32)],
            )(*round_args,
              mu_W, mu_b.reshape(1, Z), lv_W, lv_b.reshape(1, Z),
              gmu_W1, gmu_b1.reshape(1, S), gmu_W2, gmu_b2.reshape(1, G),
              glv_W1, glv_b1.reshape(1, S), glv_W2, glv_b2.reshape(1, G))

    return (mu_node, lv_node, mu_g8[0], lv_g8[0])
